# baseline (speedup 1.0000x reference)
import jax
import jax.numpy as jnp
from jax.experimental import pallas as pl
from jax.experimental.pallas import tpu as pltpu

SIZE = 32
V = SIZE * SIZE
C = 256
B = 64


def _fused_kernel(a_ref, x_ref, g_ref, lp_ref, idx_ref, lpv_ref):
    a = a_ref[...].reshape(1, C)
    x = x_ref[...].reshape(C, V)
    xc = jax.lax.dot_general(a, x, (((1,), (0,)), ((), ())))
    m = jnp.max(xc)
    lse = jnp.log(jnp.sum(jnp.exp(xc - m))) + m
    lp = xc - lse
    lp_ref[...] = lp.reshape(1, 1, V)
    s = lp + g_ref[...].reshape(1, V)
    smax = jnp.max(s)
    iota = jax.lax.broadcasted_iota(jnp.int32, (1, V), 1)
    idx = jnp.min(jnp.where(s == smax, iota, V))
    idx_ref[...] = idx.reshape(1, 1, 1)
    lpv_ref[...] = jnp.sum(jnp.where(iota == idx, lp, 0.0)).reshape(1, 1, 1)


def kernel(x, embedded_a):
    ar = embedded_a.reshape(B, 1, C)
    g = jax.random.gumbel(jax.random.key(42), (B, 1, V), dtype=jnp.float32)
    lp, idx, lpv = pl.pallas_call(
        _fused_kernel,
        grid=(B,),
        in_specs=[
            pl.BlockSpec((1, 1, C), lambda b: (b, 0, 0)),
            pl.BlockSpec((1, C, SIZE, SIZE), lambda b: (b, 0, 0, 0)),
            pl.BlockSpec((1, 1, V), lambda b: (b, 0, 0)),
        ],
        out_specs=[
            pl.BlockSpec((1, 1, V), lambda b: (b, 0, 0)),
            pl.BlockSpec((1, 1, 1), lambda b: (b, 0, 0)),
            pl.BlockSpec((1, 1, 1), lambda b: (b, 0, 0)),
        ],
        out_shape=[
            jax.ShapeDtypeStruct((B, 1, V), jnp.float32),
            jax.ShapeDtypeStruct((B, 1, 1), jnp.int32),
            jax.ShapeDtypeStruct((B, 1, 1), jnp.float32),
        ],
        compiler_params=pltpu.CompilerParams(
            dimension_semantics=("arbitrary",),
        ),
    )(ar, x, g)
    idx = idx[:, 0, 0]
    arg_lst = jnp.stack([idx % SIZE, idx // SIZE], axis=-1)
    return (arg_lst, lpv[:, 0, 0], lp.reshape(B, V))


# speedup vs baseline: 2.1149x; 2.1149x over previous
import jax
import jax.numpy as jnp
from jax.experimental import pallas as pl
from jax.experimental.pallas import tpu as pltpu

SIZE = 32
V = SIZE * SIZE
C = 256
B = 64


def _fused_kernel(a_ref, x_ref, g_ref, lp_ref, idx_ref, lpv_ref):
    X = x_ref[...].reshape(V, C)
    a = a_ref[...].reshape(C, 1)
    col = jax.lax.dot_general(X, a, (((1,), (0,)), ((), ())))
    xc = col.reshape(1, V)
    m = jnp.max(xc)
    lse = jnp.log(jnp.sum(jnp.exp(xc - m))) + m
    lp = xc - lse
    lp_ref[...] = lp.reshape(1, 1, V)
    s = lp + g_ref[...].reshape(1, V)
    smax = jnp.max(s)
    iota = jax.lax.broadcasted_iota(jnp.int32, (1, V), 1)
    idx = jnp.min(jnp.where(s == smax, iota, V))
    idx_ref[...] = idx.reshape(1, 1, 1)
    lpv_ref[...] = jnp.sum(jnp.where(iota == idx, lp, 0.0)).reshape(1, 1, 1)


def kernel(x, embedded_a):
    xt = x.transpose(0, 2, 3, 1).reshape(B, V, C)
    ar = embedded_a.reshape(B, 1, C)
    g = jax.random.gumbel(jax.random.key(42), (B, 1, V), dtype=jnp.float32)
    lp, idx, lpv = pl.pallas_call(
        _fused_kernel,
        grid=(B,),
        in_specs=[
            pl.BlockSpec((1, 1, C), lambda b: (b, 0, 0)),
            pl.BlockSpec((1, V, C), lambda b: (b, 0, 0)),
            pl.BlockSpec((1, 1, V), lambda b: (b, 0, 0)),
        ],
        out_specs=[
            pl.BlockSpec((1, 1, V), lambda b: (b, 0, 0)),
            pl.BlockSpec((1, 1, 1), lambda b: (b, 0, 0)),
            pl.BlockSpec((1, 1, 1), lambda b: (b, 0, 0)),
        ],
        out_shape=[
            jax.ShapeDtypeStruct((B, 1, V), jnp.float32),
            jax.ShapeDtypeStruct((B, 1, 1), jnp.int32),
            jax.ShapeDtypeStruct((B, 1, 1), jnp.float32),
        ],
        compiler_params=pltpu.CompilerParams(
            dimension_semantics=("arbitrary",),
        ),
    )(ar, xt, g)
    idx = idx[:, 0, 0]
    arg_lst = jnp.stack([idx % SIZE, idx // SIZE], axis=-1)
    return (arg_lst, lpv[:, 0, 0], lp.reshape(B, V))


# speedup vs baseline: 12.1725x; 5.7556x over previous
import jax
import jax.numpy as jnp
from jax.experimental import pallas as pl
from jax.experimental.pallas import tpu as pltpu

SIZE = 32
V = SIZE * SIZE
C = 256
B = 64
NB = 8


def _fused_kernel(a_ref, x_ref, g_ref, lp_ref, idx_ref, lpv_ref):
    rows = []
    for i in range(NB):
        Xi = x_ref[i]
        ai = a_ref[i, :].reshape(1, C)
        rows.append(jax.lax.dot_general(
            ai, Xi, (((1,), (1,)), ((), ()))))
    xc = jnp.concatenate(rows, axis=0)
    m = jnp.max(xc, axis=1, keepdims=True)
    lse = jnp.log(jnp.sum(jnp.exp(xc - m), axis=1, keepdims=True)) + m
    lp = xc - lse
    lp_ref[...] = lp
    s = lp + g_ref[...]
    smax = jnp.max(s, axis=1, keepdims=True)
    iota = jax.lax.broadcasted_iota(jnp.int32, (NB, V), 1)
    idx = jnp.min(jnp.where(s == smax, iota, V), axis=1, keepdims=True)
    idx_ref[...] = idx
    lpv_ref[...] = jnp.sum(jnp.where(iota == idx, lp, 0.0),
                           axis=1, keepdims=True)


def kernel(x, embedded_a):
    xt = x.transpose(0, 2, 3, 1).reshape(B, V, C)
    g = jax.random.gumbel(jax.random.key(42), (B, V), dtype=jnp.float32)
    lp, idx, lpv = pl.pallas_call(
        _fused_kernel,
        grid=(B // NB,),
        in_specs=[
            pl.BlockSpec((NB, C), lambda b: (b, 0)),
            pl.BlockSpec((NB, V, C), lambda b: (b, 0, 0)),
            pl.BlockSpec((NB, V), lambda b: (b, 0)),
        ],
        out_specs=[
            pl.BlockSpec((NB, V), lambda b: (b, 0)),
            pl.BlockSpec((NB, 1), lambda b: (b, 0)),
            pl.BlockSpec((NB, 1), lambda b: (b, 0)),
        ],
        out_shape=[
            jax.ShapeDtypeStruct((B, V), jnp.float32),
            jax.ShapeDtypeStruct((B, 1), jnp.int32),
            jax.ShapeDtypeStruct((B, 1), jnp.float32),
        ],
        compiler_params=pltpu.CompilerParams(
            dimension_semantics=("arbitrary",),
        ),
    )(embedded_a, xt, g)
    idx = idx[:, 0]
    arg_lst = jnp.stack([idx % SIZE, idx // SIZE], axis=-1)
    return (arg_lst, lpv[:, 0], lp)
